# Initial kernel scaffold; baseline (speedup 1.0000x reference)
import jax
import jax.numpy as jnp
from jax.experimental import pallas as pl


def kernel(x, rbf, sbf, idx_kj, idx_ji, params):
    raise NotImplementedError("write your pallas kernel here")


# speedup vs baseline: 3.4456x; 3.4456x over previous
import functools

import jax
import jax.numpy as jnp
from jax import lax
from jax.experimental import pallas as pl
from jax.experimental.pallas import tpu as pltpu
from jax.experimental.pallas import tpu_sc as plsc


def _silu(v):
    return v * jax.nn.sigmoid(v)


def _dot(a, b):
    return jnp.dot(a, b, preferred_element_type=jnp.float32)


def _pre_body(x_ref, rbf_ref, wji, bji, wkj, bkj, wr1, wr2, wdn,
              xji_ref, t0_ref, t1_ref, t2_ref, t3_ref):
    xb = x_ref[...]
    xji_ref[...] = _silu(_dot(xb, wji[...]) + bji[...])
    xkj = _silu(_dot(xb, wkj[...]) + bkj[...])
    r = _dot(_dot(rbf_ref[...], wr1[...]), wr2[...])
    t = _silu(_dot(xkj * r, wdn[...]))
    t0_ref[...] = t[:, 0:16]
    t1_ref[...] = t[:, 16:32]
    t2_ref[...] = t[:, 32:48]
    t3_ref[...] = t[:, 48:64]


def _run_pre(x, rbf, p):
    E, H = x.shape
    NR = rbf.shape[1]
    I = p['W_down'].shape[1]
    B = p['W_rbf1'].shape[1]
    BE = 2000
    assert E % BE == 0 and I == 64
    grid = (E // BE,)
    full = lambda shp: pl.BlockSpec(shp, lambda i: (0, 0))
    cspec = pl.BlockSpec((BE, 16), lambda i: (i, 0))
    cshape = jax.ShapeDtypeStruct((E, 16), jnp.float32)
    return pl.pallas_call(
        _pre_body,
        grid=grid,
        in_specs=[
            pl.BlockSpec((BE, H), lambda i: (i, 0)),
            pl.BlockSpec((BE, NR), lambda i: (i, 0)),
            full((H, H)), full((1, H)), full((H, H)), full((1, H)),
            full((NR, B)), full((B, H)), full((H, I)),
        ],
        out_specs=[pl.BlockSpec((BE, H), lambda i: (i, 0)),
                   cspec, cspec, cspec, cspec],
        out_shape=[jax.ShapeDtypeStruct((E, H), jnp.float32),
                   cshape, cshape, cshape, cshape],
    )(x, rbf, p['W_ji'], p['b_ji'].reshape(1, H), p['W_kj'],
      p['b_kj'].reshape(1, H), p['W_rbf1'], p['W_rbf2'], p['W_down'])


def _s_body(sbf_ref, w1, w2, s0_ref, s1_ref, s2_ref, s3_ref):
    s = _dot(_dot(sbf_ref[...], w1[...]), w2[...])
    s0_ref[...] = s[:, 0:16]
    s1_ref[...] = s[:, 16:32]
    s2_ref[...] = s[:, 32:48]
    s3_ref[...] = s[:, 48:64]


def _run_sproj(sbf, p, TPAD):
    T, SBF = sbf.shape
    B = p['W_sbf1'].shape[1]
    BT = 2048
    assert TPAD % BT == 0
    grid = (TPAD // BT,)
    full = lambda shp: pl.BlockSpec(shp, lambda i: (0, 0))
    cspec = pl.BlockSpec((BT, 16), lambda i: (i, 0))
    cshape = jax.ShapeDtypeStruct((TPAD, 16), jnp.float32)
    return pl.pallas_call(
        _s_body,
        grid=grid,
        in_specs=[pl.BlockSpec((BT, SBF), lambda i: (jnp.minimum(i, T // BT), 0)),
                  full((SBF, B)), full((B, 64))],
        out_specs=[cspec, cspec, cspec, cspec],
        out_shape=[cshape, cshape, cshape, cshape],
    )(sbf, p['W_sbf1'], p['W_sbf2'])


def _sc_aggregate(tabs, ss, ji, kj, E):
    TPAD = ss[0].shape[0]
    NS = 16
    NC = 2
    C = 128
    NCH = TPAD // (NS * C)
    assert NCH * NS * C == TPAD and NCH >= 8
    RSC = 81920
    assert NC * RSC >= E
    RT = RSC // NS
    PART = E % RT
    assert RT % 8 == 0 and PART % 8 == 0
    ZR = 1024
    assert RT % ZR == 0
    DUMMY = RSC
    NSL = 4

    mesh = plsc.VectorSubcoreMesh(core_axis_name="c", subcore_axis_name="s",
                                  num_cores=NC, num_subcores=NS)

    @functools.partial(
        pl.kernel,
        out_type=[jax.ShapeDtypeStruct((E, 16), jnp.float32)] * 4,
        mesh=mesh,
        compiler_params=pltpu.CompilerParams(use_tc_tiling_on_sc=False,
                                             needs_layout_passes=False),
        scratch_types=[
            pltpu.VMEM_SHARED((RSC + 8, 16), jnp.float32),
            pltpu.VMEM((NSL * C,), jnp.int32),
            pltpu.VMEM((NSL * C,), jnp.int32),
            pltpu.VMEM((NSL * C,), jnp.int32),
            pltpu.VMEM((NSL * C, 16), jnp.float32),
            pltpu.VMEM((NSL * C, 16), jnp.float32),
            pltpu.VMEM((ZR, 16), jnp.float32),
            pltpu.SemaphoreType.DMA((NSL,)),
            pltpu.SemaphoreType.DMA((NSL,)),
            pltpu.SemaphoreType.DMA((NSL,)),
            pltpu.SemaphoreType.DMA((NSL,)),
            pltpu.SemaphoreType.DMA((NSL,)),
        ],
    )
    def k(t0, t1, t2, t3, s0, s1, s2, s3, ji_hbm, kj_hbm,
          g0, g1, g2, g3,
          acc, jibuf, kjbuf, relbuf, tbuf, sbuf, zbuf,
          semI, semK, semT, semS, semC):
        cid = lax.axis_index("c")
        sid = lax.axis_index("s")
        lo = cid * RSC

        def zrow(rr, carry):
            zbuf[rr, pl.ds(0, 16)] = jnp.zeros((16,), jnp.float32)
            return carry
        lax.fori_loop(0, ZR, zrow, 0)

        fstart = sid * RT
        tab_list = [t0, t1, t2, t3]
        s_list = [s0, s1, s2, s3]
        out_list = [g0, g1, g2, g3]

        for pp in range(4):
            tab_hbm = tab_list[pp]
            s_hbm = s_list[pp]
            agg_hbm = out_list[pp]

            for kz in range(RT // ZR):
                pltpu.sync_copy(zbuf, acc.at[pl.ds(fstart + kz * ZR, ZR)])
            plsc.subcore_barrier()

            def idx_cp(ch, sl):
                base = (sid * NCH + ch) * C
                off = sl * C
                return (
                    pltpu.make_async_copy(ji_hbm.at[pl.ds(base, C)],
                                          jibuf.at[pl.ds(off, C)], semI.at[sl]),
                    pltpu.make_async_copy(kj_hbm.at[pl.ds(base, C)],
                                          kjbuf.at[pl.ds(off, C)], semK.at[sl]),
                )

            def gath_cp(ch, sl):
                base = (sid * NCH + ch) * C
                off = sl * C
                return (
                    pltpu.make_async_copy(tab_hbm.at[kjbuf.at[pl.ds(off, C)]],
                                          tbuf.at[pl.ds(off, C)], semT.at[sl]),
                    pltpu.make_async_copy(s_hbm.at[pl.ds(base, C)],
                                          sbuf.at[pl.ds(off, C)], semS.at[sl]),
                )

            def scat_cp(sl):
                off = sl * C
                return pltpu.make_async_copy(
                    tbuf.at[pl.ds(off, C)],
                    acc.at[relbuf.at[pl.ds(off, C)]], semC.at[sl])

            for ch in range(NSL):
                a, b = idx_cp(ch, ch)
                a.start()
                b.start()
            for ch in range(2):
                a, b = idx_cp(ch, ch)
                a.wait()
                b.wait()
                a, b = gath_cp(ch, ch)
                a.start()
                b.start()

            def body(ch, carry):
                sl = lax.rem(ch, NSL)
                off = sl * C
                for u in range(C // 16):
                    v = jibuf[pl.ds(off + 16 * u, 16)]
                    m = (v >= lo) & (v < lo + RSC)
                    relbuf[pl.ds(off + 16 * u, 16)] = \
                        jnp.where(m, v - lo, DUMMY)

                @pl.when(ch >= 2)
                def _():
                    scat_cp(lax.rem(ch + 2, NSL)).wait()

                @pl.when(ch + 2 < NCH)
                def _():
                    sl2 = lax.rem(ch + 2, NSL)
                    a, b = idx_cp(ch + 2, sl2)
                    a.wait()
                    b.wait()
                    a, b = gath_cp(ch + 2, sl2)
                    a.start()
                    b.start()

                a, b = gath_cp(ch, sl)
                a.wait()
                b.wait()

                @pl.when(ch + NSL < NCH)
                def _():
                    a2, b2 = idx_cp(ch + NSL, sl)
                    a2.start()
                    b2.start()

                def mrow(rr, c2):
                    slr = pl.ds(0, 16)
                    tbuf[rr, slr] = tbuf[rr, slr] * sbuf[rr, slr]
                    return c2
                lax.fori_loop(off, off + C, mrow, 0)

                scat_cp(sl).start()
                return carry
            lax.fori_loop(0, NCH, body, 0)

            scat_cp((NCH - 2) % NSL).wait()
            scat_cp((NCH - 1) % NSL).wait()
            plsc.subcore_barrier()

            @pl.when(lo + fstart + RT <= E)
            def _flush_full():
                pltpu.sync_copy(acc.at[pl.ds(fstart, RT)],
                                agg_hbm.at[pl.ds(lo + fstart, RT)])

            @pl.when((lo + fstart < E) & (lo + fstart + RT > E))
            def _flush_part():
                pltpu.sync_copy(acc.at[pl.ds(fstart, PART)],
                                agg_hbm.at[pl.ds(lo + fstart, PART)])

    return k(tabs[0], tabs[1], tabs[2], tabs[3],
             ss[0], ss[1], ss[2], ss[3], ji, kj)


def _post_body(a0_ref, a1_ref, a2_ref, a3_ref, xji_ref, x_ref, wup,
               wbs1, bbs1, wbs2, bbs2, wlin, blin,
               w1a, b1a, w1b, b1b, w2a, b2a, w2b, b2b, h_ref):
    agg = jnp.concatenate([a0_ref[...], a1_ref[...], a2_ref[...], a3_ref[...]],
                          axis=1)
    h = xji_ref[...] + _silu(_dot(agg, wup[...]))
    h = h + _silu(_dot(_silu(_dot(h, wbs1[...]) + bbs1[...]), wbs2[...]) + bbs2[...])
    h = _silu(_dot(h, wlin[...]) + blin[...]) + x_ref[...]
    h = h + _silu(_dot(_silu(_dot(h, w1a[...]) + b1a[...]), w1b[...]) + b1b[...])
    h = h + _silu(_dot(_silu(_dot(h, w2a[...]) + b2a[...]), w2b[...]) + b2b[...])
    h_ref[...] = h


def _run_post(aggs, xji, x, p):
    E, H = x.shape
    I = p['W_up'].shape[0]
    BE = 2000
    grid = (E // BE,)
    full = lambda shp: pl.BlockSpec(shp, lambda i: (0, 0))
    w = full((H, H))
    b = full((1, H))
    cspec = pl.BlockSpec((BE, 16), lambda i: (i, 0))
    return pl.pallas_call(
        _post_body,
        grid=grid,
        in_specs=[
            cspec, cspec, cspec, cspec,
            pl.BlockSpec((BE, H), lambda i: (i, 0)),
            pl.BlockSpec((BE, H), lambda i: (i, 0)),
            full((I, H)),
            w, b, w, b, w, b, w, b, w, b, w, b, w, b,
        ],
        out_specs=pl.BlockSpec((BE, H), lambda i: (i, 0)),
        out_shape=jax.ShapeDtypeStruct((E, H), jnp.float32),
    )(aggs[0], aggs[1], aggs[2], aggs[3], xji, x, p['W_up'],
      p['W_bs1'], p['b_bs1'].reshape(1, H), p['W_bs2'], p['b_bs2'].reshape(1, H),
      p['W_lin'], p['b_lin'].reshape(1, H),
      p['W_as1a'], p['b_as1a'].reshape(1, H), p['W_as1b'], p['b_as1b'].reshape(1, H),
      p['W_as2a'], p['b_as2a'].reshape(1, H), p['W_as2b'], p['b_as2b'].reshape(1, H))


def kernel(x, rbf, sbf, idx_kj, idx_ji, params):
    E, H = x.shape
    T = sbf.shape[0]
    TPAD = ((T + 16 * 2048 - 1) // (16 * 2048)) * (16 * 2048)

    xji, t0, t1, t2, t3 = _run_pre(x, rbf, params)
    ss = _run_sproj(sbf, params, TPAD)
    npad = TPAD - T
    ji_p = jnp.concatenate(
        [idx_ji.astype(jnp.int32), jnp.full((npad,), E, jnp.int32)])
    kj_p = jnp.concatenate(
        [idx_kj.astype(jnp.int32), jnp.zeros((npad,), jnp.int32)])
    aggs = _sc_aggregate((t0, t1, t2, t3), ss, ji_p, kj_p, E)
    return _run_post(aggs, xji, x, params)


# speedup vs baseline: 4.5827x; 1.3300x over previous
import functools

import jax
import jax.numpy as jnp
from jax import lax
from jax.experimental import pallas as pl
from jax.experimental.pallas import tpu as pltpu
from jax.experimental.pallas import tpu_sc as plsc


def _silu(v):
    return v * jax.nn.sigmoid(v)


def _dot(a, b):
    return jnp.dot(a, b, preferred_element_type=jnp.float32)


def _pre_body(x_ref, rbf_ref, wji, bji, wkj, bkj, wr1, wr2, wdn,
              xji_ref, tab_ref):
    xb = x_ref[...]
    xji_ref[...] = _silu(_dot(xb, wji[...]) + bji[...])
    xkj = _silu(_dot(xb, wkj[...]) + bkj[...])
    r = _dot(_dot(rbf_ref[...], wr1[...]), wr2[...])
    tab_ref[...] = _silu(_dot(xkj * r, wdn[...]))


def _run_pre(x, rbf, p):
    E, H = x.shape
    NR = rbf.shape[1]
    I = p['W_down'].shape[1]
    B = p['W_rbf1'].shape[1]
    BE = 2000
    assert E % BE == 0 and I == 64
    grid = (E // BE,)
    full = lambda shp: pl.BlockSpec(shp, lambda i: (0, 0))
    return pl.pallas_call(
        _pre_body,
        grid=grid,
        in_specs=[
            pl.BlockSpec((BE, H), lambda i: (i, 0)),
            pl.BlockSpec((BE, NR), lambda i: (i, 0)),
            full((H, H)), full((1, H)), full((H, H)), full((1, H)),
            full((NR, B)), full((B, H)), full((H, I)),
        ],
        out_specs=[pl.BlockSpec((BE, H), lambda i: (i, 0)),
                   pl.BlockSpec((BE, I), lambda i: (i, 0))],
        out_shape=[jax.ShapeDtypeStruct((E, H), jnp.float32),
                   jax.ShapeDtypeStruct((E, I), jnp.float32)],
    )(x, rbf, p['W_ji'], p['b_ji'].reshape(1, H), p['W_kj'],
      p['b_kj'].reshape(1, H), p['W_rbf1'], p['W_rbf2'], p['W_down'])


def _s_body(sbf_ref, w1, w2, s_ref):
    s_ref[...] = _dot(_dot(sbf_ref[...], w1[...]), w2[...])


def _run_sproj(sbf, p, TPAD):
    T, SBF = sbf.shape
    B = p['W_sbf1'].shape[1]
    BT = 2048
    assert TPAD % BT == 0
    grid = (TPAD // BT,)
    full = lambda shp: pl.BlockSpec(shp, lambda i: (0, 0))
    return pl.pallas_call(
        _s_body,
        grid=grid,
        in_specs=[pl.BlockSpec((BT, SBF), lambda i: (jnp.minimum(i, T // BT), 0)),
                  full((SBF, B)), full((B, 64))],
        out_specs=pl.BlockSpec((BT, 64), lambda i: (i, 0)),
        out_shape=jax.ShapeDtypeStruct((TPAD, 64), jnp.float32),
    )(sbf, p['W_sbf1'], p['W_sbf2'])


def _sc_aggregate(tab4, sarr, ji, kj, E):
    TPAD = sarr.shape[0]
    NS = 16
    NC = 2
    C = 128
    NCH = TPAD // (NS * C)
    assert NCH * NS * C == TPAD and NCH >= 8
    RSC = 81920
    assert NC * RSC >= E
    RT = RSC // NS
    PART = E % RT
    assert RT % 8 == 0 and PART % 8 == 0
    ZR = 1024
    assert RT % ZR == 0
    DUMMY = RSC
    NSL = 4

    mesh = plsc.VectorSubcoreMesh(core_axis_name="c", subcore_axis_name="s",
                                  num_cores=NC, num_subcores=NS)

    @functools.partial(
        pl.kernel,
        out_type=[jax.ShapeDtypeStruct((E, 16), jnp.float32)] * 4,
        mesh=mesh,
        compiler_params=pltpu.CompilerParams(use_tc_tiling_on_sc=False,
                                             needs_layout_passes=False),
        scratch_types=[
            pltpu.VMEM_SHARED((RSC + 8, 16), jnp.float32),
            pltpu.VMEM((NSL * C,), jnp.int32),
            pltpu.VMEM((NSL * C,), jnp.int32),
            pltpu.VMEM((NSL * C,), jnp.int32),
            pltpu.VMEM((NSL * C,), jnp.int32),
            pltpu.VMEM((NSL * C, 16), jnp.float32),
            pltpu.VMEM((NSL * C, 16), jnp.float32),
            pltpu.VMEM((ZR, 16), jnp.float32),
            pltpu.SemaphoreType.DMA((NSL,)),
            pltpu.SemaphoreType.DMA((NSL,)),
            pltpu.SemaphoreType.DMA((NSL,)),
            pltpu.SemaphoreType.DMA((NSL,)),
            pltpu.SemaphoreType.DMA((NSL,)),
        ],
    )
    def k(tab_hbm, s_hbm, ji_hbm, kj_hbm,
          g0, g1, g2, g3,
          acc, jibuf, kjbuf, relbuf, kidxbuf, tbuf, sbuf, zbuf,
          semI, semK, semT, semS, semC):
        cid = lax.axis_index("c")
        sid = lax.axis_index("s")
        lo = cid * RSC

        def zrow(rr, carry):
            zbuf[rr, pl.ds(0, 16)] = jnp.zeros((16,), jnp.float32)
            return carry
        lax.fori_loop(0, ZR, zrow, 0)

        fstart = sid * RT
        out_list = [g0, g1, g2, g3]

        for pp in range(4):
            agg_hbm = out_list[pp]

            for kz in range(RT // ZR):
                pltpu.sync_copy(zbuf, acc.at[pl.ds(fstart + kz * ZR, ZR)])
            plsc.subcore_barrier()

            def idx_cp(ch, sl):
                base = (sid * NCH + ch) * C
                off = sl * C
                return (
                    pltpu.make_async_copy(ji_hbm.at[pl.ds(base, C)],
                                          jibuf.at[pl.ds(off, C)], semI.at[sl]),
                    pltpu.make_async_copy(kj_hbm.at[pl.ds(base, C)],
                                          kjbuf.at[pl.ds(off, C)], semK.at[sl]),
                )

            def gath_cp(ch, sl):
                base = (sid * NCH + ch) * C
                off = sl * C
                return (
                    pltpu.make_async_copy(tab_hbm.at[kidxbuf.at[pl.ds(off, C)]],
                                          tbuf.at[pl.ds(off, C)], semT.at[sl]),
                    pltpu.make_async_copy(
                        s_hbm.at[pl.ds(base, C), pl.ds(16 * pp, 16)],
                        sbuf.at[pl.ds(off, C)], semS.at[sl]),
                )

            def kidx_compute(sl):
                off = sl * C
                for u in range(C // 16):
                    kv = kjbuf[pl.ds(off + 16 * u, 16)]
                    kidxbuf[pl.ds(off + 16 * u, 16)] = kv * 4 + pp

            def scat_cp(sl):
                off = sl * C
                return pltpu.make_async_copy(
                    tbuf.at[pl.ds(off, C)],
                    acc.at[relbuf.at[pl.ds(off, C)]], semC.at[sl])

            for ch in range(NSL):
                a, b = idx_cp(ch, ch)
                a.start()
                b.start()
            for ch in range(2):
                a, b = idx_cp(ch, ch)
                a.wait()
                b.wait()
                kidx_compute(ch)
                a, b = gath_cp(ch, ch)
                a.start()
                b.start()

            def body(ch, carry):
                sl = lax.rem(ch, NSL)
                off = sl * C
                for u in range(C // 16):
                    v = jibuf[pl.ds(off + 16 * u, 16)]
                    m = (v >= lo) & (v < lo + RSC)
                    relbuf[pl.ds(off + 16 * u, 16)] = \
                        jnp.where(m, v - lo, DUMMY)

                @pl.when(ch >= 2)
                def _():
                    scat_cp(lax.rem(ch + 2, NSL)).wait()

                @pl.when(ch + 2 < NCH)
                def _():
                    sl2 = lax.rem(ch + 2, NSL)
                    a, b = idx_cp(ch + 2, sl2)
                    a.wait()
                    b.wait()
                    kidx_compute(sl2)
                    a, b = gath_cp(ch + 2, sl2)
                    a.start()
                    b.start()

                a, b = gath_cp(ch, sl)
                a.wait()
                b.wait()

                @pl.when(ch + NSL < NCH)
                def _():
                    a2, b2 = idx_cp(ch + NSL, sl)
                    a2.start()
                    b2.start()

                def mrow(rr, c2):
                    slr = pl.ds(0, 16)
                    for uu in range(8):
                        tbuf[off + 8 * rr + uu, slr] = \
                            tbuf[off + 8 * rr + uu, slr] * sbuf[off + 8 * rr + uu, slr]
                    return c2
                lax.fori_loop(0, C // 8, mrow, 0)

                scat_cp(sl).start()
                return carry
            lax.fori_loop(0, NCH, body, 0)

            scat_cp((NCH - 2) % NSL).wait()
            scat_cp((NCH - 1) % NSL).wait()
            plsc.subcore_barrier()

            @pl.when(lo + fstart + RT <= E)
            def _flush_full():
                pltpu.sync_copy(acc.at[pl.ds(fstart, RT)],
                                agg_hbm.at[pl.ds(lo + fstart, RT)])

            @pl.when((lo + fstart < E) & (lo + fstart + RT > E))
            def _flush_part():
                pltpu.sync_copy(acc.at[pl.ds(fstart, PART)],
                                agg_hbm.at[pl.ds(lo + fstart, PART)])

    return k(tab4, sarr, ji, kj)


def _post_body(a0_ref, a1_ref, a2_ref, a3_ref, xji_ref, x_ref, wup,
               wbs1, bbs1, wbs2, bbs2, wlin, blin,
               w1a, b1a, w1b, b1b, w2a, b2a, w2b, b2b, h_ref):
    agg = jnp.concatenate([a0_ref[...], a1_ref[...], a2_ref[...], a3_ref[...]],
                          axis=1)
    h = xji_ref[...] + _silu(_dot(agg, wup[...]))
    h = h + _silu(_dot(_silu(_dot(h, wbs1[...]) + bbs1[...]), wbs2[...]) + bbs2[...])
    h = _silu(_dot(h, wlin[...]) + blin[...]) + x_ref[...]
    h = h + _silu(_dot(_silu(_dot(h, w1a[...]) + b1a[...]), w1b[...]) + b1b[...])
    h = h + _silu(_dot(_silu(_dot(h, w2a[...]) + b2a[...]), w2b[...]) + b2b[...])
    h_ref[...] = h


def _run_post(aggs, xji, x, p):
    E, H = x.shape
    I = p['W_up'].shape[0]
    BE = 2000
    grid = (E // BE,)
    full = lambda shp: pl.BlockSpec(shp, lambda i: (0, 0))
    w = full((H, H))
    b = full((1, H))
    cspec = pl.BlockSpec((BE, 16), lambda i: (i, 0))
    return pl.pallas_call(
        _post_body,
        grid=grid,
        in_specs=[
            cspec, cspec, cspec, cspec,
            pl.BlockSpec((BE, H), lambda i: (i, 0)),
            pl.BlockSpec((BE, H), lambda i: (i, 0)),
            full((I, H)),
            w, b, w, b, w, b, w, b, w, b, w, b, w, b,
        ],
        out_specs=pl.BlockSpec((BE, H), lambda i: (i, 0)),
        out_shape=jax.ShapeDtypeStruct((E, H), jnp.float32),
    )(aggs[0], aggs[1], aggs[2], aggs[3], xji, x, p['W_up'],
      p['W_bs1'], p['b_bs1'].reshape(1, H), p['W_bs2'], p['b_bs2'].reshape(1, H),
      p['W_lin'], p['b_lin'].reshape(1, H),
      p['W_as1a'], p['b_as1a'].reshape(1, H), p['W_as1b'], p['b_as1b'].reshape(1, H),
      p['W_as2a'], p['b_as2a'].reshape(1, H), p['W_as2b'], p['b_as2b'].reshape(1, H))


def kernel(x, rbf, sbf, idx_kj, idx_ji, params):
    E, H = x.shape
    T = sbf.shape[0]
    TPAD = ((T + 16 * 2048 - 1) // (16 * 2048)) * (16 * 2048)

    xji, table = _run_pre(x, rbf, params)
    sarr = _run_sproj(sbf, params, TPAD)
    npad = TPAD - T
    ji_p = jnp.concatenate(
        [idx_ji.astype(jnp.int32), jnp.full((npad,), E, jnp.int32)])
    kj_p = jnp.concatenate(
        [idx_kj.astype(jnp.int32), jnp.zeros((npad,), jnp.int32)])
    aggs = _sc_aggregate(table.reshape(E * 4, 16), sarr, ji_p, kj_p, E)
    return _run_post(aggs, xji, x, params)
